# Initial kernel scaffold; baseline (speedup 1.0000x reference)
import jax
import jax.numpy as jnp
from jax.experimental import pallas as pl


def kernel(x_road, x_datetime, road_table, datetime_table):
    raise NotImplementedError("write your pallas kernel here")


# speedup vs baseline: 2.3051x; 2.3051x over previous
import functools

import jax
import jax.numpy as jnp
from jax import lax
from jax.experimental import pallas as pl
from jax.experimental.pallas import tpu as pltpu
from jax.experimental.pallas import tpu_sc as plsc

N = 16384
P = 20
D = 32
R = N * P
NC, NS = 2, 16
NW = NC * NS
ROWS_W = R // NW
CHUNK = 1024
G = 128
NG = CHUNK // G
NCHUNK = ROWS_W // CHUNK


def _sc_embed(road_table, datetime_table, idx_road, idx_dt):
  mesh = plsc.VectorSubcoreMesh(core_axis_name="c", subcore_axis_name="s")

  @functools.partial(
      pl.kernel,
      mesh=mesh,
      compiler_params=pltpu.CompilerParams(use_tc_tiling_on_sc=False),
      out_type=jax.ShapeDtypeStruct((R, 2, D), jnp.float32),
      scratch_types=[
          pltpu.VMEM((NG, G), jnp.int32),
          pltpu.VMEM((NG, G), jnp.int32),
          pltpu.VMEM((CHUNK, D), jnp.float32),
          pltpu.VMEM((CHUNK, D), jnp.float32),
          pltpu.SemaphoreType.DMA,
      ],
  )
  def k(road_hbm, dt_hbm, idxr_hbm, idxd_hbm, out_hbm,
        idxr_v, idxd_v, rbuf, dbuf, sem):
    wid = lax.axis_index("s") * NC + lax.axis_index("c")

    def body(ci, carry):
      row0 = pl.multiple_of(wid * ROWS_W + ci * CHUNK, CHUNK)
      g0 = pl.multiple_of(row0 // G, NG)
      pltpu.sync_copy(idxr_hbm.at[pl.ds(g0, NG)], idxr_v)
      pltpu.sync_copy(idxd_hbm.at[pl.ds(g0, NG)], idxd_v)
      cps = []
      for g in range(NG):
        cps.append(pltpu.async_copy(
            road_hbm.at[idxr_v.at[g]], rbuf.at[pl.ds(g * G, G)], sem))
        cps.append(pltpu.async_copy(
            dt_hbm.at[idxd_v.at[g]], dbuf.at[pl.ds(g * G, G)], sem))
      for cp in cps:
        cp.wait()
      pltpu.sync_copy(rbuf, out_hbm.at[pl.ds(row0, CHUNK), 0])
      pltpu.sync_copy(dbuf, out_hbm.at[pl.ds(row0, CHUNK), 1])
      return carry

    lax.fori_loop(0, NCHUNK, body, 0)

  return k(road_table, datetime_table, idx_road, idx_dt)


def kernel(x_road, x_datetime, road_table, datetime_table):
  xr = x_road.reshape(N).astype(jnp.int32)
  idx_road = jnp.broadcast_to(xr[:, None], (N, P)).reshape(R // G, G)
  idx_dt = x_datetime.reshape(R // G, G).astype(jnp.int32)
  out = _sc_embed(road_table, datetime_table, idx_road, idx_dt)
  return out.reshape(N, P, 2 * D)
